# baseline (speedup 1.0000x reference)
import functools

import jax
import jax.numpy as jnp
from jax import lax
from jax.experimental import pallas as pl
from jax.experimental.pallas import tpu as pltpu
from jax.experimental.pallas import tpu_sc as plsc

_B = 4096
_NN = 13
_NC = 26
_NF = _NN + _NC
_D = 128
_CARDP = 1001
_STRIDE = 1008
_TROWS = (_NC + 1) * _STRIDE
_RTOT = _B * _NF
_SPW = 104
_NSTREAM = _RTOT // _SPW
_NWORK = 32
_KPW = _NSTREAM // _NWORK
_BPW = _B // _NWORK
_NBUF = 6


def _bake_body(lnb_ref, pw_ref, pb_ref, fnum_ref, cat_ref, fcat_ref, out_ref):
    i = pl.program_id(0)

    @pl.when(i == 0)
    def _():
        out_ref[0:16, :] = (lnb_ref[...] * pw_ref[...] + pb_ref[...]
                            + fnum_ref[...])
        out_ref[16:, :] = jnp.zeros((_STRIDE - 16, _D), jnp.float32)

    @pl.when(i > 0)
    def _():
        out_ref[0:_CARDP, :] = cat_ref[0] + fcat_ref[0]
        out_ref[_CARDP:, :] = jnp.zeros((_STRIDE - _CARDP, _D), jnp.float32)


def _bake(lnb_b, pw, pb, fnum, cat_tables, fcat):
    return pl.pallas_call(
        _bake_body,
        grid=(_NC + 1,),
        in_specs=[
            pl.BlockSpec((16, _D), lambda i: (0, 0)),
            pl.BlockSpec((16, _D), lambda i: (0, 0)),
            pl.BlockSpec((16, _D), lambda i: (0, 0)),
            pl.BlockSpec((16, _D), lambda i: (0, 0)),
            pl.BlockSpec((1, _CARDP, _D),
                         lambda i: (jnp.maximum(i - 1, 0), 0, 0)),
            pl.BlockSpec((1, 1, _D),
                         lambda i: (jnp.minimum(12 + i, _NF - 1), 0, 0)),
        ],
        out_specs=pl.BlockSpec((_STRIDE, _D), lambda i: (i, 0)),
        out_shape=jax.ShapeDtypeStruct((_TROWS, _D), jnp.float32),
    )(lnb_b, pw, pb, fnum, cat_tables, fcat)


_RPW = _RTOT // _NWORK


def _sc_body(xext_hbm, table_hbm, out_hbm, xc_v, idx_v, *rest):
    bufs = list(rest[:_NBUF])
    gsem, wsem = rest[_NBUF], rest[_NBUF + 1]
    wid = lax.axis_index("s") * 2 + lax.axis_index("c")
    k0 = wid * _KPW
    r0 = wid * _RPW
    pltpu.sync_copy(xext_hbm.at[pl.ds(r0, _RPW)], xc_v)

    lane = lax.iota(jnp.int32, 16)
    def grp_fn(g, carry):
        pos = lane + (r0 + g * 16)
        f = lax.shift_right_logical(pos, 12)
        base = jnp.where(f < _NN, f, (f - (_NN - 1)) * _STRIDE)
        idx_v[pl.ds(g * 16, 16)] = base + xc_v[pl.ds(g * 16, 16)]
        return carry
    lax.fori_loop(0, _RPW // 16, grp_fn, 0)

    for s in range(_NBUF):
        pltpu.async_copy(table_hbm.at[idx_v.at[pl.ds(s * _SPW, _SPW)]],
                         bufs[s], gsem.at[s])

    def outer(t, carry):
        handles = []
        for s in range(_NBUF):
            k = t * _NBUF + s
            pltpu.make_async_copy(
                table_hbm.at[idx_v.at[pl.ds(k * _SPW, _SPW)]], bufs[s],
                gsem.at[s]).wait()
            handles.append(pltpu.async_copy(
                bufs[s], out_hbm.at[pl.ds((k0 + k) * _SPW, _SPW)],
                wsem.at[s]))
        for s in range(_NBUF):
            handles[s].wait()
            kn = (t + 1) * _NBUF + s

            @pl.when(kn < _KPW)
            def _(s=s, kn=kn):
                pltpu.async_copy(
                    table_hbm.at[idx_v.at[pl.ds(kn * _SPW, _SPW)]], bufs[s],
                    gsem.at[s])
        return carry

    lax.fori_loop(0, _KPW // _NBUF, outer, 0)


def _sc_gather(xext, flat_table):
    mesh = plsc.VectorSubcoreMesh(core_axis_name="c", subcore_axis_name="s")
    fn = functools.partial(
        pl.kernel,
        mesh=mesh,
        out_type=jax.ShapeDtypeStruct((_RTOT, _D), jnp.float32),
        scratch_types=(
            [pltpu.VMEM((_RPW,), jnp.int32),
             pltpu.VMEM((_RPW,), jnp.int32)]
            + [pltpu.VMEM((_SPW, _D), jnp.float32) for _ in range(_NBUF)]
            + [pltpu.SemaphoreType.DMA((_NBUF,)),
               pltpu.SemaphoreType.DMA((_NBUF,))]
        ),
    )(_sc_body)
    return fn(xext, flat_table)


def kernel(x_num, x_cat, ln_w, ln_b, proj_w, proj_b, cat_tables, feat_id):
    del x_num, ln_w
    f32 = jnp.float32
    lnb_b = jnp.broadcast_to(jnp.pad(ln_b.astype(f32), (0, 3))[:, None],
                             (16, _D))
    pw = jnp.pad(proj_w.astype(f32), ((0, 3), (0, 0)))
    pb = jnp.pad(proj_b.astype(f32), ((0, 3), (0, 0)))
    fnum = jnp.pad(feat_id[:_NN].astype(f32), ((0, 3), (0, 0)))
    fcat = feat_id.astype(f32).reshape(_NF, 1, _D)

    flat_table = _bake(lnb_b, pw, pb, fnum, cat_tables.astype(f32), fcat)

    xext = jnp.concatenate(
        [jnp.zeros((_NN * _B,), jnp.int32),
         x_cat.astype(jnp.int32).T.reshape(_NC * _B)])
    out_flat = _sc_gather(xext, flat_table)
    return out_flat.reshape(_NF, _B, _D).transpose(1, 0, 2)


# speedup vs baseline: 5.1170x; 5.1170x over previous
import functools

import jax
import jax.numpy as jnp
from jax import lax
from jax.experimental import pallas as pl
from jax.experimental.pallas import tpu as pltpu
from jax.experimental.pallas import tpu_sc as plsc

_B = 4096
_NN = 13
_NC = 26
_NF = _NN + _NC
_D = 128
_CARDP = 1001
_STRIDE = 1008
_TROWS = (_NC + 1) * _STRIDE
_RTOT = _B * _NF
_SPW = 104
_NSTREAM = _RTOT // _SPW
_NWORK = 32
_KPW = _NSTREAM // _NWORK
_BPW = _B // _NWORK
_NBUF = 4


def _bake_body(lnb_ref, pw_ref, pb_ref, fnum_ref, cat_ref, fcat_ref, out_ref):
    i = pl.program_id(0)

    @pl.when(i == 0)
    def _():
        out_ref[0:16, :] = (lnb_ref[...] * pw_ref[...] + pb_ref[...]
                            + fnum_ref[...])
        out_ref[16:, :] = jnp.zeros((_STRIDE - 16, _D), jnp.float32)

    @pl.when(i > 0)
    def _():
        out_ref[0:_CARDP, :] = cat_ref[0] + fcat_ref[0]
        out_ref[_CARDP:, :] = jnp.zeros((_STRIDE - _CARDP, _D), jnp.float32)


def _bake(lnb_b, pw, pb, fnum, cat_tables, fcat):
    return pl.pallas_call(
        _bake_body,
        grid=(_NC + 1,),
        in_specs=[
            pl.BlockSpec((16, _D), lambda i: (0, 0)),
            pl.BlockSpec((16, _D), lambda i: (0, 0)),
            pl.BlockSpec((16, _D), lambda i: (0, 0)),
            pl.BlockSpec((16, _D), lambda i: (0, 0)),
            pl.BlockSpec((1, _CARDP, _D),
                         lambda i: (jnp.maximum(i - 1, 0), 0, 0)),
            pl.BlockSpec((1, 1, _D),
                         lambda i: (jnp.minimum(12 + i, _NF - 1), 0, 0)),
        ],
        out_specs=pl.BlockSpec((_STRIDE, _D), lambda i: (i, 0)),
        out_shape=jax.ShapeDtypeStruct((_TROWS, _D), jnp.float32),
    )(lnb_b, pw, pb, fnum, cat_tables, fcat)


_CPW = _NC * _B // _NWORK
_CKPW = _CPW // _SPW
_CAT0 = _NN * _B


def _sc_body(xcat_hbm, table_hbm, out_hbm, xc_v, idx_v, ns_v, nb0, nb1,
             *rest):
    nbufs = [nb0, nb1]
    bufs = list(rest[:_NBUF])
    gsem, wsem, nsem = rest[_NBUF], rest[_NBUF + 1], rest[_NBUF + 2]
    wid = lax.axis_index("s") * 2 + lax.axis_index("c")
    p0 = wid * _CPW
    pltpu.sync_copy(xcat_hbm.at[pl.ds(p0, _CPW)], xc_v)
    pltpu.sync_copy(table_hbm.at[pl.ds(0, 16)], ns_v)

    lane = lax.iota(jnp.int32, 16)
    def grp_fn(g, carry):
        pos = lane + (p0 + g * 16)
        c = lax.shift_right_logical(pos, 12)
        idx_v[pl.ds(g * 16, 16)] = (c + 1) * _STRIDE + xc_v[pl.ds(g * 16, 16)]
        return carry
    lax.fori_loop(0, _CPW // 16, grp_fn, 0)

    for s in range(_NBUF):
        pltpu.async_copy(table_hbm.at[idx_v.at[pl.ds(s * _SPW, _SPW)]],
                         bufs[s], gsem.at[s])

    for f in range(_NN):
        pp = f % 2
        if f >= 2:
            pltpu.make_async_copy(
                nbufs[pp], out_hbm.at[pl.ds(wid * 128, 128)],
                nsem.at[pp]).wait()
        vs = [ns_v[f, pl.ds(j * 16, 16)] for j in range(8)]
        def rep_fn(rr, carry, pp=pp, vs=vs):
            for j in range(8):
                nbufs[pp][rr, pl.ds(j * 16, 16)] = vs[j]
            return carry
        lax.fori_loop(0, 128, rep_fn, 0)
        pltpu.async_copy(nbufs[pp],
                         out_hbm.at[pl.ds(f * _B + wid * 128, 128)],
                         nsem.at[pp])
    for pp in range(2):
        pltpu.make_async_copy(nbufs[pp],
                              out_hbm.at[pl.ds(wid * 128, 128)],
                              nsem.at[pp]).wait()

    def outer(t, carry):
        handles = []
        for s in range(_NBUF):
            k = t * _NBUF + s
            pltpu.make_async_copy(
                table_hbm.at[idx_v.at[pl.ds(k * _SPW, _SPW)]], bufs[s],
                gsem.at[s]).wait()
            handles.append(pltpu.async_copy(
                bufs[s], out_hbm.at[pl.ds(_CAT0 + p0 + k * _SPW, _SPW)],
                wsem.at[s]))
        for s in range(_NBUF):
            handles[s].wait()
            kn = (t + 1) * _NBUF + s

            @pl.when(kn < _CKPW)
            def _(s=s, kn=kn):
                pltpu.async_copy(
                    table_hbm.at[idx_v.at[pl.ds(kn * _SPW, _SPW)]], bufs[s],
                    gsem.at[s])
        return carry

    lax.fori_loop(0, _CKPW // _NBUF, outer, 0)


def _sc_gather(xcatf, flat_table):
    mesh = plsc.VectorSubcoreMesh(core_axis_name="c", subcore_axis_name="s")
    fn = functools.partial(
        pl.kernel,
        mesh=mesh,
        out_type=jax.ShapeDtypeStruct((_RTOT, _D), jnp.float32),
        scratch_types=(
            [pltpu.VMEM((_CPW,), jnp.int32),
             pltpu.VMEM((_CPW,), jnp.int32),
             pltpu.VMEM((16, _D), jnp.float32),
             pltpu.VMEM((128, _D), jnp.float32),
             pltpu.VMEM((128, _D), jnp.float32)]
            + [pltpu.VMEM((_SPW, _D), jnp.float32) for _ in range(_NBUF)]
            + [pltpu.SemaphoreType.DMA((_NBUF,)),
               pltpu.SemaphoreType.DMA((_NBUF,)),
               pltpu.SemaphoreType.DMA((2,))]
        ),
    )(_sc_body)
    return fn(xcatf, flat_table)


def kernel(x_num, x_cat, ln_w, ln_b, proj_w, proj_b, cat_tables, feat_id):
    del x_num, ln_w
    f32 = jnp.float32
    lnb_b = jnp.broadcast_to(jnp.pad(ln_b.astype(f32), (0, 3))[:, None],
                             (16, _D))
    pw = jnp.pad(proj_w.astype(f32), ((0, 3), (0, 0)))
    pb = jnp.pad(proj_b.astype(f32), ((0, 3), (0, 0)))
    fnum = jnp.pad(feat_id[:_NN].astype(f32), ((0, 3), (0, 0)))
    fcat = feat_id.astype(f32).reshape(_NF, 1, _D)

    flat_table = _bake(lnb_b, pw, pb, fnum, cat_tables.astype(f32), fcat)

    xcatf = x_cat.astype(jnp.int32).T.reshape(_NC * _B)
    out_flat = _sc_gather(xcatf, flat_table)
    return out_flat.reshape(_NF, _B, _D).transpose(1, 0, 2)


# speedup vs baseline: 6.2795x; 1.2272x over previous
import functools

import jax
import jax.numpy as jnp
from jax import lax
from jax.experimental import pallas as pl
from jax.experimental.pallas import tpu as pltpu
from jax.experimental.pallas import tpu_sc as plsc

_B = 4096
_NN = 13
_NC = 26
_NF = _NN + _NC
_D = 128
_CARDP = 1001
_RTOT = _B * _NF
_NWORK = 32
_NBUF = 4


def _sc_body(xcatf_hbm, tab_hbm, lnb_hbm, pw_hbm, pb_hbm, fid_hbm, out_hbm,
             xc_v, lnb_v, pw_v, pb_v, fid_v, nb0, nb1, *rest):
    nbufs = [nb0, nb1]
    bufs = list(rest[:_NBUF])
    gsem, wsem, nsem, xsem = (rest[_NBUF], rest[_NBUF + 1], rest[_NBUF + 2],
                              rest[_NBUF + 3])
    wid = lax.axis_index("s") * 2 + lax.axis_index("c")
    bb = wid * 128

    for c in range(_NC):
        pltpu.async_copy(xcatf_hbm.at[pl.ds(c * _B + bb, 128)],
                         xc_v.at[c], xsem)
    pltpu.sync_copy(lnb_hbm, lnb_v)
    pltpu.sync_copy(pw_hbm, pw_v)
    pltpu.sync_copy(pb_hbm, pb_v)
    pltpu.sync_copy(fid_hbm, fid_v)
    for c in range(_NC):
        pltpu.make_async_copy(xcatf_hbm.at[pl.ds(c * _B + bb, 128)],
                              xc_v.at[c], xsem).wait()

    for s in range(_NBUF):
        pltpu.async_copy(tab_hbm.at[s].at[xc_v.at[s]], bufs[s], gsem.at[s])

    lnb_vec = lnb_v[pl.ds(0, 16)]
    for f in range(_NN):
        pp = f % 2
        if f >= 2:
            pltpu.make_async_copy(
                nbufs[pp], out_hbm.at[pl.ds(bb, 128)], nsem.at[pp]).wait()
        lnb_s = lnb_vec[f]
        vs = [lnb_s * pw_v[pl.ds(f * _D + j * 16, 16)]
              + pb_v[pl.ds(f * _D + j * 16, 16)]
              + fid_v[pl.ds(f * _D + j * 16, 16)] for j in range(8)]
        def rep_fn(rr, carry, pp=pp, vs=vs):
            for j in range(8):
                nbufs[pp][rr, pl.ds(j * 16, 16)] = vs[j]
            return carry
        lax.fori_loop(0, 128, rep_fn, 0)
        pltpu.async_copy(nbufs[pp], out_hbm.at[pl.ds(f * _B + bb, 128)],
                         nsem.at[pp])
    for pp in range(2):
        pltpu.make_async_copy(nbufs[pp], out_hbm.at[pl.ds(bb, 128)],
                              nsem.at[pp]).wait()

    ngrp = (_NC + _NBUF - 1) // _NBUF
    for g in range(ngrp):
        for s in range(_NBUF):
            c = g * _NBUF + s
            if c >= _NC:
                break
            pltpu.make_async_copy(tab_hbm.at[c].at[xc_v.at[c]], bufs[s],
                                  gsem.at[s]).wait()
            fvs = [fid_v[pl.ds((_NN + c) * _D + j * 16, 16)]
                   for j in range(8)]
            def add_fn(rr, carry, s=s, fvs=fvs):
                for j in range(8):
                    sl = (rr, pl.ds(j * 16, 16))
                    bufs[s][sl] = bufs[s][sl] + fvs[j]
                return carry
            lax.fori_loop(0, 128, add_fn, 0)
            pltpu.async_copy(
                bufs[s], out_hbm.at[pl.ds((_NN + c) * _B + bb, 128)],
                wsem.at[s])
        for s in range(_NBUF):
            c = g * _NBUF + s
            cn = c + _NBUF
            if c >= _NC:
                break
            pltpu.make_async_copy(
                bufs[s], out_hbm.at[pl.ds((_NN + c) * _B + bb, 128)],
                wsem.at[s]).wait()
            if cn < _NC:
                pltpu.async_copy(tab_hbm.at[cn].at[xc_v.at[cn]], bufs[s],
                                 gsem.at[s])


def _sc_tokenize(xcatf, cat_tables, lnb16, pwf, pbf, fidf):
    mesh = plsc.VectorSubcoreMesh(core_axis_name="c", subcore_axis_name="s")
    fn = functools.partial(
        pl.kernel,
        mesh=mesh,
        out_type=jax.ShapeDtypeStruct((_RTOT, _D), jnp.float32),
        scratch_types=(
            [pltpu.VMEM((_NC, 128), jnp.int32),
             pltpu.VMEM((16,), jnp.float32),
             pltpu.VMEM((16 * _D,), jnp.float32),
             pltpu.VMEM((16 * _D,), jnp.float32),
             pltpu.VMEM((_NF * _D,), jnp.float32),
             pltpu.VMEM((128, _D), jnp.float32),
             pltpu.VMEM((128, _D), jnp.float32)]
            + [pltpu.VMEM((128, _D), jnp.float32) for _ in range(_NBUF)]
            + [pltpu.SemaphoreType.DMA((_NBUF,)),
               pltpu.SemaphoreType.DMA((_NBUF,)),
               pltpu.SemaphoreType.DMA((2,)),
               pltpu.SemaphoreType.DMA]
        ),
    )(_sc_body)
    return fn(xcatf, cat_tables, lnb16, pwf, pbf, fidf)


def kernel(x_num, x_cat, ln_w, ln_b, proj_w, proj_b, cat_tables, feat_id):
    del x_num, ln_w
    f32 = jnp.float32
    xcatf = x_cat.astype(jnp.int32).T.reshape(_NC * _B)
    lnb16 = jnp.pad(ln_b.astype(f32), (0, 3))
    pwf = jnp.pad(proj_w.astype(f32), ((0, 3), (0, 0))).reshape(16 * _D)
    pbf = jnp.pad(proj_b.astype(f32), ((0, 3), (0, 0))).reshape(16 * _D)
    fidf = feat_id.astype(f32).reshape(_NF * _D)

    out_flat = _sc_tokenize(xcatf, cat_tables.astype(f32), lnb16, pwf, pbf,
                            fidf)
    return out_flat.reshape(_NF, _B, _D).transpose(1, 0, 2)


# speedup vs baseline: 6.4420x; 1.0259x over previous
import functools

import jax
import jax.numpy as jnp
from jax import lax
from jax.experimental import pallas as pl
from jax.experimental.pallas import tpu as pltpu
from jax.experimental.pallas import tpu_sc as plsc

_B = 4096
_NN = 13
_NC = 26
_NF = _NN + _NC
_D = 128
_CARDP = 1001
_RTOT = _B * _NF
_NWORK = 32
_NBUF = 5


def _sc_body(xcatf_hbm, tab_hbm, lnb_hbm, pw_hbm, pb_hbm, fid_hbm, out_hbm,
             xc_v, lnb_v, pw_v, pb_v, fid_v, nb0, nb1, *rest):
    nbufs = [nb0, nb1]
    bufs = list(rest[:_NBUF])
    gsem, wsem, nsem, xsem = (rest[_NBUF], rest[_NBUF + 1], rest[_NBUF + 2],
                              rest[_NBUF + 3])
    wid = lax.axis_index("s") * 2 + lax.axis_index("c")
    bb = wid * 128

    pltpu.async_copy(xcatf_hbm.at[:, pl.ds(bb, 128)], xc_v, xsem)
    pltpu.sync_copy(lnb_hbm, lnb_v)
    pltpu.sync_copy(pw_hbm, pw_v)
    pltpu.sync_copy(pb_hbm, pb_v)
    pltpu.sync_copy(fid_hbm, fid_v)
    pltpu.make_async_copy(xcatf_hbm.at[:, pl.ds(bb, 128)], xc_v, xsem).wait()

    for s in range(_NBUF):
        pltpu.async_copy(tab_hbm.at[s].at[xc_v.at[s]], bufs[s], gsem.at[s])

    lnb_vec = lnb_v[pl.ds(0, 16)]
    for f in range(_NN):
        pp = f % 2
        if f >= 2:
            pltpu.make_async_copy(
                nbufs[pp], out_hbm.at[pl.ds(bb, 128)], nsem.at[pp]).wait()
        lnb_s = lnb_vec[f]
        vs = [lnb_s * pw_v[pl.ds(f * _D + j * 16, 16)]
              + pb_v[pl.ds(f * _D + j * 16, 16)]
              + fid_v[pl.ds(f * _D + j * 16, 16)] for j in range(8)]
        def rep_fn(rr, carry, pp=pp, vs=vs):
            for j in range(8):
                nbufs[pp][rr, pl.ds(j * 16, 16)] = vs[j]
            return carry
        lax.fori_loop(0, 128, rep_fn, 0)
        pltpu.async_copy(nbufs[pp], out_hbm.at[pl.ds(f * _B + bb, 128)],
                         nsem.at[pp])
    for pp in range(2):
        pltpu.make_async_copy(nbufs[pp], out_hbm.at[pl.ds(bb, 128)],
                              nsem.at[pp]).wait()

    ngrp = (_NC + _NBUF - 1) // _NBUF
    for g in range(ngrp):
        for s in range(_NBUF):
            c = g * _NBUF + s
            if c >= _NC:
                break
            pltpu.make_async_copy(tab_hbm.at[c].at[xc_v.at[c]], bufs[s],
                                  gsem.at[s]).wait()
            fvs = [fid_v[pl.ds((_NN + c) * _D + j * 16, 16)]
                   for j in range(8)]
            def add_fn(rr, carry, s=s, fvs=fvs):
                for j in range(8):
                    plsc.addupdate(bufs[s].at[rr, pl.ds(j * 16, 16)], fvs[j])
                return carry
            lax.fori_loop(0, 128, add_fn, 0)
            pltpu.async_copy(
                bufs[s], out_hbm.at[pl.ds((_NN + c) * _B + bb, 128)],
                wsem.at[s])
        for s in range(_NBUF):
            c = g * _NBUF + s
            cn = c + _NBUF
            if c >= _NC:
                break
            pltpu.make_async_copy(
                bufs[s], out_hbm.at[pl.ds((_NN + c) * _B + bb, 128)],
                wsem.at[s]).wait()
            if cn < _NC:
                pltpu.async_copy(tab_hbm.at[cn].at[xc_v.at[cn]], bufs[s],
                                 gsem.at[s])


def _sc_tokenize(xcat2, cat_tables, lnb16, pwf, pbf, fidf):
    mesh = plsc.VectorSubcoreMesh(core_axis_name="c", subcore_axis_name="s")
    fn = functools.partial(
        pl.kernel,
        mesh=mesh,
        out_type=jax.ShapeDtypeStruct((_RTOT, _D), jnp.float32),
        scratch_types=(
            [pltpu.VMEM((_NC, 128), jnp.int32),
             pltpu.VMEM((16,), jnp.float32),
             pltpu.VMEM((16 * _D,), jnp.float32),
             pltpu.VMEM((16 * _D,), jnp.float32),
             pltpu.VMEM((_NF * _D,), jnp.float32),
             pltpu.VMEM((128, _D), jnp.float32),
             pltpu.VMEM((128, _D), jnp.float32)]
            + [pltpu.VMEM((128, _D), jnp.float32) for _ in range(_NBUF)]
            + [pltpu.SemaphoreType.DMA((_NBUF,)),
               pltpu.SemaphoreType.DMA((_NBUF,)),
               pltpu.SemaphoreType.DMA((2,)),
               pltpu.SemaphoreType.DMA]
        ),
    )(_sc_body)
    return fn(xcat2, cat_tables, lnb16, pwf, pbf, fidf)


def kernel(x_num, x_cat, ln_w, ln_b, proj_w, proj_b, cat_tables, feat_id):
    del x_num, ln_w
    f32 = jnp.float32
    xcat2 = x_cat.astype(jnp.int32).T
    lnb16 = jnp.pad(ln_b.astype(f32), (0, 3))
    pwf = jnp.pad(proj_w.astype(f32), ((0, 3), (0, 0))).reshape(16 * _D)
    pbf = jnp.pad(proj_b.astype(f32), ((0, 3), (0, 0))).reshape(16 * _D)
    fidf = feat_id.astype(f32).reshape(_NF * _D)

    out_flat = _sc_tokenize(xcat2, cat_tables.astype(f32), lnb16, pwf, pbf,
                            fidf)
    return out_flat.reshape(_NF, _B, _D).transpose(1, 0, 2)
